# Initial kernel scaffold; baseline (speedup 1.0000x reference)
import jax
import jax.numpy as jnp
from jax.experimental import pallas as pl


def kernel(boards):
    raise NotImplementedError("write your pallas kernel here")


# speedup vs baseline: 29.8058x; 29.8058x over previous
import functools

import jax
import jax.numpy as jnp
import numpy as np
from jax.experimental import pallas as pl


def _rf(sq):
    return sq // 8, sq % 8


def _build_tables():
    diag = np.zeros((64, 64), dtype=bool)
    fr = np.zeros((64, 64), dtype=bool)
    for i in range(64):
        ri, fi = _rf(i)
        for j in range(64):
            rj, fj = _rf(j)
            if (ri - fi == rj - fj) or (ri + fi == rj + fj):
                diag[i, j] = True
            if ri == rj or fi == fj:
                fr[i, j] = True

    def leaper(deltas, self_conn):
        m = np.zeros((64, 64), dtype=bool)
        for i in range(64):
            ri, fi = _rf(i)
            if self_conn:
                m[i, i] = True
            for dr, df in deltas:
                rj, fj = ri + dr, fi + df
                if 0 <= rj < 8 and 0 <= fj < 8:
                    m[i, rj * 8 + fj] = True
        return m

    def pawn(direction):
        m = np.zeros((64, 64), dtype=bool)
        for i in range(64):
            ri, fi = _rf(i)
            for df in (-1, 1):
                rj, fj = ri + direction, fi + df
                if 0 <= rj < 8 and 0 <= fj < 8:
                    m[i, rj * 8 + fj] = True
        return m

    knight = leaper([(-2, -1), (-2, 1), (-1, -2), (-1, 2),
                     (1, -2), (1, 2), (2, -1), (2, 1)], True)
    king = leaper([(-1, -1), (-1, 0), (-1, 1), (0, -1),
                   (0, 1), (1, -1), (1, 0), (1, 1)], True)
    nonslide = np.zeros((13, 64, 64), dtype=bool)
    nonslide[0] = pawn(1)
    nonslide[1] = knight
    nonslide[5] = king
    nonslide[6] = pawn(-1)
    nonslide[7] = knight
    nonslide[11] = king
    slide = np.zeros((13, 64, 64), dtype=bool)
    slide[2] = diag
    slide[3] = fr
    slide[4] = diag | fr
    slide[8] = diag
    slide[9] = fr
    slide[10] = diag | fr

    alignedf = np.zeros((1, 64 * 64), dtype=np.float32)
    btw = np.zeros((64, 64 * 64), dtype=np.float32)
    for i in range(64):
        ri, fi = _rf(i)
        for j in range(64):
            if i == j:
                continue
            rj, fj = _rf(j)
            dr, df = rj - ri, fj - fi
            aligned = (df == 0) or (dr == 0) or (abs(dr) == abs(df))
            if not aligned:
                continue
            q = i * 64 + j
            alignedf[0, q] = 1.0
            sr = (dr > 0) - (dr < 0)
            sf = (df > 0) - (df < 0)
            cr, cf = ri + sr, fi + sf
            while (cr, cf) != (rj, fj):
                btw[cr * 8 + cf, q] = 1.0
                cr += sr
                cf += sf

    ecomb = np.zeros((64 * 13, 64 * 64), dtype=np.float32)
    for i in range(64):
        for t in range(13):
            ecomb[i * 13 + t, i * 64:(i + 1) * 64] = (
                2.0 * nonslide[t, i].astype(np.float32)
                + slide[t, i].astype(np.float32))

    g = np.zeros((64, 64 * 13), dtype=np.float32)
    for c in range(64 * 13):
        g[c // 13, c] = 1.0
    tpat = (np.arange(64 * 13, dtype=np.float32) % 13).reshape(1, -1)

    return (btw, alignedf, g, tpat, ecomb.astype(jnp.bfloat16))


_BTW, _ALIGNEDF, _G, _TPAT, _ECOMB = _build_tables()


def _mask_body(planes_ref, btw_ref, al_ref, g_ref, tpat_ref, ec_ref,
               ray_ref, att_ref):
    x = planes_ref[...]
    tot = x[:, 0:64]
    best = tot
    idx = jnp.zeros_like(tot)
    for t in range(1, 12):
        sl = x[:, t * 64:(t + 1) * 64]
        m = sl > best
        best = jnp.where(m, sl, best)
        idx = jnp.where(m, jnp.full_like(idx, float(t)), idx)
        tot = tot + sl
    occ = tot > 0.5
    occf = occ.astype(jnp.float32)
    ptype = jnp.where(occ, idx, jnp.full_like(idx, 12.0))

    blocked = jnp.dot(occf, btw_ref[...], preferred_element_type=jnp.float32)
    ray = (al_ref[...] > 0.5) & (blocked < 0.5)

    pexp = jnp.dot(ptype, g_ref[...], preferred_element_type=jnp.float32)
    oh = (pexp == tpat_ref[...]).astype(jnp.bfloat16)
    v = jnp.dot(oh, ec_ref[...], preferred_element_type=jnp.float32)
    ns = v > 1.5
    slb = (v == 1.0) | (v == 3.0)
    att = ns | (slb & ray)

    ray_ref[...] = ray
    att_ref[...] = att


@functools.partial(jax.jit, static_argnames=("bt", "interpret"))
def _run(planes2, bt, interpret=False):
    b = planes2.shape[0]
    grid = (b // bt,)
    ray2, att2 = pl.pallas_call(
        _mask_body,
        grid=grid,
        in_specs=[
            pl.BlockSpec((bt, 768), lambda i: (i, 0)),
            pl.BlockSpec((64, 4096), lambda i: (0, 0)),
            pl.BlockSpec((1, 4096), lambda i: (0, 0)),
            pl.BlockSpec((64, 832), lambda i: (0, 0)),
            pl.BlockSpec((1, 832), lambda i: (0, 0)),
            pl.BlockSpec((832, 4096), lambda i: (0, 0)),
        ],
        out_specs=[
            pl.BlockSpec((bt, 4096), lambda i: (i, 0)),
            pl.BlockSpec((bt, 4096), lambda i: (i, 0)),
        ],
        out_shape=[
            jax.ShapeDtypeStruct((b, 4096), jnp.bool_),
            jax.ShapeDtypeStruct((b, 4096), jnp.bool_),
        ],
        interpret=interpret,
    )(planes2, _BTW, _ALIGNEDF, _G, _TPAT, _ECOMB)
    return ray2, att2


def kernel(boards, *, bt=128, interpret=False):
    b = boards.shape[0]
    planes2 = boards.reshape(b, 18 * 64)[:, :768]
    ray2, att2 = _run(planes2, bt, interpret)
    return ray2.reshape(b, 64, 64), att2.reshape(b, 64, 64)
